# Initial kernel scaffold; baseline (speedup 1.0000x reference)
import jax
import jax.numpy as jnp
from jax.experimental import pallas as pl


def kernel(pred_logits, pred_boxes, target_sizes):
    raise NotImplementedError("write your pallas kernel here")


# speedup vs baseline: 9.1809x; 9.1809x over previous
import functools

import jax
import jax.numpy as jnp
from jax import lax
from jax.experimental import pallas as pl
from jax.experimental.pallas import tpu as pltpu
from jax.experimental.pallas import tpu_sc as plsc

NUM_Q = 900
NUM_C = 91
N_REAL = NUM_Q * NUM_C
N_PAD = 81920
NV = N_PAD // 16
K_OUT = 300
K_PAD = 304
CAP = 4096
NW = 32
ROWS_PER_W = 4

_I32 = jnp.int32
_U32 = jnp.uint32


def _lane():
    return lax.iota(_I32, 16)


def _popcount(mask):
    return plsc.all_reduce_population_count(mask)[0]


def _clear_hist(hist_v):
    zeros = jnp.zeros((16,), _I32)

    def body(i, _):
        hist_v[pl.ds(i * 16, 16)] = zeros
        return 0

    lax.fori_loop(0, 256, body, 0)


def _lanesum(hist_v, b):
    v = hist_v[pl.ds(b * 16, 16)]
    return jnp.sum(v)


def _bin_walk(hist_v, g0):
    def cond(s):
        _, g, c = s
        return g + c < K_OUT

    def body(s):
        b, g, c = s
        b2 = b - 1
        return b2, g + c, _lanesum(hist_v, b2)

    b0 = jnp.int32(255)
    s = lax.while_loop(cond, body, (b0, g0, _lanesum(hist_v, b0)))
    return s


def _keys_of(bits):
    neg = bits >= jnp.uint32(0x80000000)
    return jnp.where(neg, ~bits, bits | jnp.uint32(0x80000000))


def _tk_body(bits_hbm, boxes_hbm, wh_hbm, scores_hbm, labels_hbm,
             boxout_hbm, qidx_hbm, row_v, hist_v, ckey_v, cidx_v,
             fkey_v, fidx_v, skey_v, sidx_v, sc_v, lb_v, qi_v, bx_v,
             brow_v, wh_v):
    wid = lax.axis_index("s") * 2 + lax.axis_index("c")
    lane = _lane()
    ones = jnp.ones((16,), _I32)

    def do_row(r_i, _):
        r = wid * ROWS_PER_W + r_i
        pltpu.sync_copy(bits_hbm.at[r], row_v)
        pltpu.sync_copy(boxes_hbm.at[r], brow_v)
        pltpu.sync_copy(wh_hbm.at[r], wh_v)

        _clear_hist(hist_v)

        def hist0(i, _):
            bits = row_v[pl.ds(i * 16, 16)]
            key = _keys_of(bits)
            row_v[pl.ds(i * 16, 16)] = key
            binv = (key >> jnp.uint32(24)).astype(_I32)
            plsc.addupdate_scatter(hist_v, [(binv << 4) | lane], ones)
            return 0

        lax.fori_loop(0, NV, hist0, 0)

        b, g, c = _bin_walk(hist_v, jnp.int32(0))
        t_lo = b.astype(_U32) << jnp.uint32(24)
        n_ge = g + c

        def refine_cond(s):
            lvl, _, _, n = s
            return (n > CAP) & (lvl < 3)

        def refine_body(s):
            lvl, t, g_in, _ = s
            lvl2 = lvl + 1
            shift = (jnp.uint32(24) - jnp.uint32(8) * lvl2.astype(_U32))
            hi = shift + jnp.uint32(8)
            _clear_hist(hist_v)

            def histl(i, _):
                key = row_v[pl.ds(i * 16, 16)]
                match = (key >> hi) == (t >> hi)
                binv = ((key >> shift) & jnp.uint32(0xFF)).astype(_I32)
                plsc.addupdate_scatter(hist_v, [(binv << 4) | lane], ones,
                                       mask=match)
                return 0

            lax.fori_loop(0, NV, histl, 0)
            b2, g2, c2 = _bin_walk(hist_v, g_in)
            t2 = t | (b2.astype(_U32) << shift)
            return lvl2, t2, g2, g2 + c2

        lvl_f, t_lo, g, n_ge = lax.while_loop(
            refine_cond, refine_body, (jnp.int32(0), t_lo, g, n_ge))

        exact_from_row = n_ge > CAP

        def compact_exact(src_ref, sidx_ref, nvec, m_src, t, e_need,
                          dst_key, dst_idx, from_row):
            def body(i, s):
                off, eq_taken = s
                key = src_ref[pl.ds(i * 16, 16)]
                if from_row:
                    idxv = lane + i * 16
                    valid = jnp.full((16,), True)
                else:
                    idxv = sidx_ref[pl.ds(i * 16, 16)]
                    valid = (lane + i * 16) < m_src
                m_gt = (key > t) & valid
                m_eq = (key == t) & valid
                eqc = plsc.cumsum(m_eq.astype(_I32))
                take = m_eq & ((eq_taken + eqc) <= e_need)
                m = m_gt | take
                plsc.store_compressed(dst_key.at[pl.ds(off, 16)],
                                      key.astype(_I32), mask=m)
                plsc.store_compressed(dst_idx.at[pl.ds(off, 16)], idxv,
                                      mask=m)
                return off + _popcount(m), eq_taken + _popcount(take)

            lax.fori_loop(0, nvec, body, (jnp.int32(0), jnp.int32(0)))

        def from_row_exact(_):
            compact_exact(row_v, None, jnp.int32(NV), jnp.int32(0), t_lo,
                          K_OUT - g, fkey_v, fidx_v, True)
            return 0

        def via_candidates(_):
            def cbody(i, off):
                key = row_v[pl.ds(i * 16, 16)]
                idxv = lane + i * 16
                m = key >= t_lo
                plsc.store_compressed(ckey_v.at[pl.ds(off, 16)], key, mask=m)
                plsc.store_compressed(cidx_v.at[pl.ds(off, 16)], idxv,
                                      mask=m)
                return off + _popcount(m)

            m_cand = lax.fori_loop(0, NV, cbody, jnp.int32(0))
            mv = (m_cand + 15) // 16

            t = jnp.uint32(0)
            g2 = jnp.int32(0)
            c2 = jnp.int32(0)
            for l in range(4):
                shift = jnp.uint32(24 - 8 * l)
                _clear_hist(hist_v)

                def histc(i, _, shift=shift, t_ref=None):
                    key = ckey_v[pl.ds(i * 16, 16)]
                    valid = (lane + i * 16) < m_cand
                    if l == 0:
                        match = valid
                    else:
                        hi = shift + jnp.uint32(8)
                        match = valid & ((key >> hi) == (t_cur >> hi))
                    binv = ((key >> shift) & jnp.uint32(0xFF)).astype(_I32)
                    plsc.addupdate_scatter(hist_v, [(binv << 4) | lane],
                                           ones, mask=match)
                    return 0

                t_cur = t
                lax.fori_loop(0, mv, histc, 0)
                b3, g2, c2 = _bin_walk(hist_v, g2)
                t = t | (b3.astype(_U32) << shift)

            e_need = K_OUT - g2
            compact_exact(ckey_v, cidx_v, mv, m_cand, t, e_need,
                          fkey_v, fidx_v, False)
            return 0

        lax.cond(exact_from_row, from_row_exact, via_candidates, 0)

        pad_pos = (lane & 3) + 300
        pad_m = lane < 4
        plsc.store_scatter(fkey_v, [pad_pos], jnp.zeros((16,), _I32),
                           mask=pad_m)
        plsc.store_scatter(fidx_v, [pad_pos],
                           jnp.full((16,), 0x7FFFFFFF, _I32), mask=pad_m)
        skey_v[pl.ds(288, 16)] = jnp.zeros((16,), _I32)
        sidx_v[pl.ds(288, 16)] = jnp.zeros((16,), _I32)

        def rank_group(v, _):
            kv = plsc.bitcast(fkey_v[pl.ds(v * 16, 16)], _U32)
            iv = fidx_v[pl.ds(v * 16, 16)]

            def rbody(j, rank):
                jb = jnp.full((16,), 1, _I32) * j
                kj = plsc.bitcast(plsc.load_gather(fkey_v, [jb]), _U32)
                ij = plsc.load_gather(fidx_v, [jb])
                beat = (kj > kv) | ((kj == kv) & (ij < iv))
                return rank + beat.astype(_I32)

            rank = lax.fori_loop(0, K_OUT, rbody, jnp.zeros((16,), _I32))
            valid = (lane + v * 16) < K_OUT
            plsc.store_scatter(skey_v, [rank],
                               plsc.bitcast(kv, _I32), mask=valid)
            plsc.store_scatter(sidx_v, [rank], iv, mask=valid)
            return 0

        lax.fori_loop(0, K_PAD // 16, rank_group, 0)

        wvec = wh_v[pl.ds(0, 16)]
        hvec = wh_v[pl.ds(16, 16)]
        lane4 = lane * 4

        def emit_group(g_i, _):
            key = plsc.bitcast(skey_v[pl.ds(g_i * 16, 16)], _U32)
            msb = key >= jnp.uint32(0x80000000)
            bits = jnp.where(msb, key ^ jnp.uint32(0x80000000), ~key)
            x = plsc.bitcast(bits, jnp.float32)
            score = 1.0 / (1.0 + jnp.exp(-x))
            idx = sidx_v[pl.ds(g_i * 16, 16)]
            q = lax.div(idx, jnp.int32(NUM_C))
            label = idx - q * NUM_C
            q4 = q * 4
            cx = plsc.load_gather(brow_v, [q4])
            cy = plsc.load_gather(brow_v, [q4 + 1])
            w = plsc.load_gather(brow_v, [q4 + 2])
            h = plsc.load_gather(brow_v, [q4 + 3])
            hw = 0.5 * w
            hh = 0.5 * h
            sc_v[pl.ds(g_i * 16, 16)] = score
            lb_v[pl.ds(g_i * 16, 16)] = label
            qi_v[pl.ds(g_i * 16, 16)] = q
            base = g_i * 64
            plsc.store_scatter(bx_v, [lane4 + base], (cx - hw) * wvec)
            plsc.store_scatter(bx_v, [lane4 + base + 1], (cy - hh) * hvec)
            plsc.store_scatter(bx_v, [lane4 + base + 2], (cx + hw) * wvec)
            plsc.store_scatter(bx_v, [lane4 + base + 3], (cy + hh) * hvec)
            return 0

        lax.fori_loop(0, K_PAD // 16, emit_group, 0)

        pltpu.sync_copy(sc_v, scores_hbm.at[r])
        pltpu.sync_copy(lb_v, labels_hbm.at[r])
        pltpu.sync_copy(qi_v, qidx_hbm.at[r])
        pltpu.sync_copy(bx_v, boxout_hbm.at[r])
        return 0

    lax.fori_loop(0, ROWS_PER_W, do_row, 0)


@jax.jit
def _topk_sc(bits, boxes_flat, wh):
    b = bits.shape[0]
    mesh = plsc.VectorSubcoreMesh(core_axis_name="c", subcore_axis_name="s",
                                  num_cores=2, num_subcores=16)
    out_type = (
        jax.ShapeDtypeStruct((b, K_PAD), jnp.float32),
        jax.ShapeDtypeStruct((b, K_PAD), jnp.int32),
        jax.ShapeDtypeStruct((b, K_PAD * 4), jnp.float32),
        jax.ShapeDtypeStruct((b, K_PAD), jnp.int32),
    )
    scratch = [
        pltpu.VMEM((N_PAD,), _U32),
        pltpu.VMEM((4096,), _I32),
        pltpu.VMEM((CAP + 16,), _U32),
        pltpu.VMEM((CAP + 16,), _I32),
        pltpu.VMEM((K_PAD,), _I32),
        pltpu.VMEM((K_PAD,), _I32),
        pltpu.VMEM((K_PAD,), _I32),
        pltpu.VMEM((K_PAD,), _I32),
        pltpu.VMEM((K_PAD,), jnp.float32),
        pltpu.VMEM((K_PAD,), _I32),
        pltpu.VMEM((K_PAD,), _I32),
        pltpu.VMEM((K_PAD * 4,), jnp.float32),
        pltpu.VMEM((NUM_Q * 4,), jnp.float32),
        pltpu.VMEM((32,), jnp.float32),
    ]
    f = pl.kernel(_tk_body, out_type=out_type, mesh=mesh,
                  scratch_types=scratch,
                  compiler_params=pltpu.CompilerParams(
                      needs_layout_passes=False))
    return f(bits, boxes_flat, wh)


def kernel(pred_logits, pred_boxes, target_sizes):
    b, q, c = pred_logits.shape
    flat = pred_logits.reshape(b, q * c)
    pad = jnp.full((b, N_PAD - N_REAL), -jnp.inf, jnp.float32)
    bits = lax.bitcast_convert_type(
        jnp.concatenate([flat, pad], axis=1), _U32)
    boxes_flat = pred_boxes.reshape(b, q * 4)
    ts = target_sizes.astype(jnp.float32)
    wv = jnp.broadcast_to(ts[:, 1:2], (b, 16))
    hv = jnp.broadcast_to(ts[:, 0:1], (b, 16))
    wh = jnp.concatenate([wv, hv], axis=1)
    scores, labels, boxes, qidx = _topk_sc(bits, boxes_flat, wh)
    boxes = boxes[:, : K_OUT * 4].reshape(b, K_OUT, 4)
    return (scores[:, :K_OUT], labels[:, :K_OUT], boxes, qidx[:, :K_OUT])
